# baseline (speedup 1.0000x reference)
import jax
import jax.numpy as jnp
from jax.experimental import pallas as pl
from jax.experimental.pallas import tpu as pltpu

B = 4096
K = 64
NCHUNK = 8
CHUNK = B // NCHUNK


def _body(in_hbm, xui_ref, u_v, w_v, *sems):
    copies = []
    for g in range(NCHUNK):
        rows = pl.ds(g * CHUNK, CHUNK)
        cu = pltpu.make_async_copy(in_hbm.at[0, rows], u_v.at[rows], sems[2 * g])
        cw = pltpu.make_async_copy(in_hbm.at[1, rows], w_v.at[rows], sems[2 * g + 1])
        cu.start()
        cw.start()
        copies.append((cu, cw))
    for g in range(NCHUNK):
        cu, cw = copies[g]
        cu.wait()
        cw.wait()
        rows = pl.ds(g * CHUNK, CHUNK)
        xui_ref[rows] = jnp.sum(u_v[rows, :] * w_v[rows, :], axis=1)


def kernel(inputs):
    xui = pl.pallas_call(
        _body,
        in_specs=[pl.BlockSpec(memory_space=pltpu.MemorySpace.HBM)],
        out_specs=pl.BlockSpec(memory_space=pltpu.MemorySpace.VMEM),
        out_shape=jax.ShapeDtypeStruct((B,), jnp.float32),
        scratch_shapes=[
            pltpu.VMEM((B, K), jnp.float32),
            pltpu.VMEM((B, K), jnp.float32),
        ] + [pltpu.SemaphoreType.DMA] * (2 * NCHUNK),
    )(inputs)
    return (xui, inputs[0], inputs[1])


# speedup vs baseline: 1.0632x; 1.0632x over previous
import jax
import jax.numpy as jnp
from jax.experimental import pallas as pl
from jax.experimental.pallas import tpu as pltpu

B = 4096
K = 64
NCHUNK = 4
CHUNK = B // NCHUNK


def _body(in_hbm, xui_ref, s_v, *sems):
    copies = []
    for g in range(NCHUNK):
        rows = pl.ds(g * CHUNK, CHUNK)
        c = pltpu.make_async_copy(in_hbm.at[:, rows], s_v.at[:, rows], sems[g])
        c.start()
        copies.append(c)
    for g in range(NCHUNK):
        copies[g].wait()
        rows = pl.ds(g * CHUNK, CHUNK)
        xui_ref[rows] = jnp.sum(s_v[0, rows, :] * s_v[1, rows, :], axis=1)


def kernel(inputs):
    xui = pl.pallas_call(
        _body,
        in_specs=[pl.BlockSpec(memory_space=pltpu.MemorySpace.HBM)],
        out_specs=pl.BlockSpec(memory_space=pltpu.MemorySpace.VMEM),
        out_shape=jax.ShapeDtypeStruct((B,), jnp.float32),
        scratch_shapes=[
            pltpu.VMEM((2, B, K), jnp.float32),
        ] + [pltpu.SemaphoreType.DMA] * NCHUNK,
    )(inputs)
    return (xui, inputs[0], inputs[1])


# speedup vs baseline: 1.0655x; 1.0022x over previous
import jax
import jax.numpy as jnp
from jax.experimental import pallas as pl
from jax.experimental.pallas import tpu as pltpu

B = 4096
K = 64
NCHUNK = 4
CHUNK = B // NCHUNK


def _body(in_hbm, xui_ref, u_v, w_v, *sems):
    copies = []
    for g in range(NCHUNK):
        rows = pl.ds(g * CHUNK, CHUNK)
        cu = pltpu.make_async_copy(in_hbm.at[0, rows], u_v.at[rows], sems[2 * g])
        cw = pltpu.make_async_copy(in_hbm.at[1, rows], w_v.at[rows], sems[2 * g + 1])
        cu.start()
        cw.start()
        copies.append((cu, cw))
    for g in range(NCHUNK):
        cu, cw = copies[g]
        cu.wait()
        cw.wait()
        rows = pl.ds(g * CHUNK, CHUNK)
        xui_ref[rows] = jnp.sum(u_v[rows, :] * w_v[rows, :], axis=1)


def kernel(inputs):
    xui = pl.pallas_call(
        _body,
        in_specs=[pl.BlockSpec(memory_space=pltpu.MemorySpace.HBM)],
        out_specs=pl.BlockSpec(memory_space=pltpu.MemorySpace.VMEM),
        out_shape=jax.ShapeDtypeStruct((B,), jnp.float32),
        scratch_shapes=[
            pltpu.VMEM((B, K), jnp.float32),
            pltpu.VMEM((B, K), jnp.float32),
        ] + [pltpu.SemaphoreType.DMA] * (2 * NCHUNK),
    )(inputs)
    return (xui, inputs[0], inputs[1])
